# Initial kernel scaffold; baseline (speedup 1.0000x reference)
import functools

import numpy as np
import jax
import jax.numpy as jnp
from jax import lax
from jax.experimental import pallas as pl
from jax.experimental.pallas import tpu as pltpu
from jax.experimental.pallas import tpu_sc as plsc

_NC = 2
_NS = 16
_L = 16
_NW = _NC * _NS

_SHAPE = (2, 8192, 4096)
_N = _SHAPE[0] * _SHAPE[1] * _SHAPE[2]
_PER_W = _N // _NW
_B = 16384
_NBLK = _PER_W // _B

_TBL = 4097
_TBL_PAD = 4112

_MAGIC = np.float32(12582912.0)


def _build_table() -> np.ndarray:
    xs = np.linspace(-8.0, 8.0, num=_TBL).astype(np.float32)
    ys = 1.0 / (1.0 + np.exp(-xs.astype(np.float64)))
    yq = np.rint(ys * 256.0)
    tbl = (yq / 256.0).astype(np.float32)
    out = np.zeros((_TBL_PAD,), dtype=np.float32)
    out[:_TBL] = tbl
    return out


_TABLE = _build_table()

_mesh = plsc.VectorSubcoreMesh(
    core_axis_name="c", subcore_axis_name="s",
    num_cores=_NC, num_subcores=_NS,
)


@functools.partial(
    pl.kernel,
    out_type=jax.ShapeDtypeStruct((_N,), jnp.float32),
    mesh=_mesh,
    scratch_types=[
        pltpu.VMEM((_TBL_PAD,), jnp.float32),
        pltpu.VMEM((_B,), jnp.float32),
        pltpu.VMEM((_B,), jnp.float32),
    ],
)
def _lut_kernel(x_hbm, tbl_hbm, out_hbm, tbl_v, in_v, out_v):
    wid = lax.axis_index("s") * _NC + lax.axis_index("c")
    base = wid * _PER_W
    pltpu.sync_copy(tbl_hbm, tbl_v)

    def block(g, carry):
        off = base + g * _B
        pltpu.sync_copy(x_hbm.at[pl.ds(off, _B)], in_v)

        def vec(i, c):
            xv = in_v[pl.ds(i * _L, _L)]
            t = xv * np.float32(256.0)
            r = (t + _MAGIC) - _MAGIC
            r = jnp.minimum(jnp.maximum(r, np.float32(-2048.0)),
                            np.float32(2048.0))
            idx = (r + np.float32(2048.0)).astype(jnp.int32)
            out_v[pl.ds(i * _L, _L)] = plsc.load_gather(tbl_v, [idx])
            return c

        lax.fori_loop(0, _B // _L, vec, 0, unroll=8)
        pltpu.sync_copy(out_v, out_hbm.at[pl.ds(off, _B)])
        return carry

    lax.fori_loop(0, _NBLK, block, 0)


def kernel(x):
    y = _lut_kernel(x.reshape(-1), _TABLE)
    return y.reshape(_SHAPE)


# speedup vs baseline: 196.2655x; 196.2655x over previous
import functools

import numpy as np
import jax
import jax.numpy as jnp
from jax import lax
from jax.experimental import pallas as pl
from jax.experimental.pallas import tpu as pltpu
from jax.experimental.pallas import tpu_sc as plsc

_NC = 2
_NS = 16
_L = 16
_NW = _NC * _NS

_SHAPE = (2, 8192, 4096)
_N = _SHAPE[0] * _SHAPE[1] * _SHAPE[2]
_PER_W = _N // _NW
_B = 16384
_NBLK = _PER_W // _B

_TBL = 4097
_TBL_PAD = 4112

_MAGIC = np.float32(12582912.0)


def _build_table() -> np.ndarray:
    xs = np.linspace(-8.0, 8.0, num=_TBL).astype(np.float32)
    ys = 1.0 / (1.0 + np.exp(-xs.astype(np.float64)))
    yq = np.rint(ys * 256.0)
    tbl = (yq / 256.0).astype(np.float32)
    out = np.zeros((_TBL_PAD,), dtype=np.float32)
    out[:_TBL] = tbl
    return out


_TABLE = _build_table()


@functools.cache
def _get_lut_kernel():
    mesh = plsc.VectorSubcoreMesh(
        core_axis_name="c", subcore_axis_name="s",
        num_cores=_NC, num_subcores=_NS,
    )

    @functools.partial(
        pl.kernel,
        out_type=jax.ShapeDtypeStruct((_N,), jnp.float32),
        mesh=mesh,
        compiler_params=pltpu.CompilerParams(needs_layout_passes=False),
        scratch_types=[
            pltpu.VMEM((_TBL_PAD,), jnp.float32),
            pltpu.VMEM((_B,), jnp.float32),
            pltpu.VMEM((_B,), jnp.float32),
        ],
    )
    def _lut_kernel(x_hbm, tbl_hbm, out_hbm, tbl_v, in_v, out_v):
        wid = lax.axis_index("s") * _NC + lax.axis_index("c")
        base = wid * _PER_W
        pltpu.sync_copy(tbl_hbm, tbl_v)

        def block(g, carry):
            off = base + g * _B
            pltpu.sync_copy(x_hbm.at[pl.ds(off, _B)], in_v)

            def vec(i, c):
                xv = in_v[pl.ds(i * _L, _L)]
                t = xv * np.float32(256.0)
                r = (t + _MAGIC) - _MAGIC
                r = jnp.minimum(jnp.maximum(r, np.float32(-2048.0)),
                                np.float32(2048.0))
                idx = (r + np.float32(2048.0)).astype(jnp.int32)
                out_v[pl.ds(i * _L, _L)] = plsc.load_gather(tbl_v, [idx])
                return c

            lax.fori_loop(0, _B // _L, vec, 0, unroll=8)
            pltpu.sync_copy(out_v, out_hbm.at[pl.ds(off, _B)])
            return carry

        lax.fori_loop(0, _NBLK, block, 0)

    return _lut_kernel


def kernel(x):
    y = _get_lut_kernel()(x.reshape(-1), _TABLE)
    return y.reshape(_SHAPE)


# speedup vs baseline: 225.5382x; 1.1491x over previous
import functools

import numpy as np
import jax
import jax.numpy as jnp
from jax import lax
from jax.experimental import pallas as pl
from jax.experimental.pallas import tpu as pltpu
from jax.experimental.pallas import tpu_sc as plsc

_NC = 2
_NS = 16
_L = 16
_NW = _NC * _NS

_SHAPE = (2, 8192, 4096)
_N = _SHAPE[0] * _SHAPE[1] * _SHAPE[2]
_PER_W = _N // _NW
_B = 16384
_NBLK = _PER_W // _B

_TBL = 4097
_TBL_PAD = 4112

_MAGIC = np.float32(12582912.0)


def _build_table() -> np.ndarray:
    xs = np.linspace(-8.0, 8.0, num=_TBL).astype(np.float32)
    ys = 1.0 / (1.0 + np.exp(-xs.astype(np.float64)))
    yq = np.rint(ys * 256.0)
    tbl = (yq / 256.0).astype(np.float32)
    out = np.zeros((_TBL_PAD,), dtype=np.float32)
    out[:_TBL] = tbl
    return out


_TABLE = _build_table()


@functools.cache
def _get_lut_kernel():
    mesh = plsc.VectorSubcoreMesh(
        core_axis_name="c", subcore_axis_name="s",
        num_cores=_NC, num_subcores=_NS,
    )

    @functools.partial(
        pl.kernel,
        out_type=jax.ShapeDtypeStruct((_N,), jnp.float32),
        mesh=mesh,
        compiler_params=pltpu.CompilerParams(needs_layout_passes=False),
        scratch_types=[
            pltpu.VMEM((_TBL_PAD,), jnp.float32),
            pltpu.VMEM((_B,), jnp.float32),
            pltpu.VMEM((_B,), jnp.float32),
            pltpu.VMEM((_B,), jnp.float32),
            pltpu.VMEM((_B,), jnp.float32),
            pltpu.SemaphoreType.DMA,
            pltpu.SemaphoreType.DMA,
            pltpu.SemaphoreType.DMA,
            pltpu.SemaphoreType.DMA,
        ],
    )
    def _lut_kernel(x_hbm, tbl_hbm, out_hbm, tbl_v,
                    in0, in1, out0, out1, si0, si1, so0, so1):
        wid = lax.axis_index("s") * _NC + lax.axis_index("c")
        base = wid * _PER_W
        ins, outs = (in0, in1), (out0, out1)
        sis, sos = (si0, si1), (so0, so1)

        def in_copy(g, b):
            off = base + g * _B
            return pltpu.make_async_copy(
                x_hbm.at[pl.ds(off, _B)], ins[b], sis[b])

        def out_copy(g, b):
            off = base + g * _B
            return pltpu.make_async_copy(
                outs[b], out_hbm.at[pl.ds(off, _B)], sos[b])

        def compute(in_v, out_v):
            def vec(i, c):
                xv = in_v[pl.ds(i * _L, _L)]
                u = xv * np.float32(256.0) + _MAGIC
                v = u - (_MAGIC - np.float32(2048.0))
                v = jnp.minimum(jnp.maximum(v, np.float32(0.0)),
                                np.float32(4096.0))
                idx = v.astype(jnp.int32)
                out_v[pl.ds(i * _L, _L)] = plsc.load_gather(tbl_v, [idx])
                return c

            lax.fori_loop(0, _B // _L, vec, 0, unroll=8)

        in_copy(0, 0).start()
        pltpu.sync_copy(tbl_hbm, tbl_v)

        def pair(g2, carry):
            for b in range(2):
                g = g2 * 2 + b

                @pl.when(g + 1 < _NBLK)
                def _():
                    in_copy(g + 1, 1 - b).start()

                in_copy(g, b).wait()

                @pl.when(g >= 2)
                def _():
                    out_copy(g - 2, b).wait()

                compute(ins[b], outs[b])
                out_copy(g, b).start()
            return carry

        lax.fori_loop(0, _NBLK // 2, pair, 0)
        out_copy(_NBLK - 2, 0).wait()
        out_copy(_NBLK - 1, 1).wait()

    return _lut_kernel


def kernel(x):
    y = _get_lut_kernel()(x.reshape(-1), _TABLE)
    return y.reshape(_SHAPE)


# speedup vs baseline: 887.8517x; 3.9366x over previous
import functools

import numpy as np
import jax
import jax.numpy as jnp
from jax import lax
from jax.experimental import pallas as pl
from jax.experimental.pallas import tpu as pltpu
from jax.experimental.pallas import tpu_sc as plsc

_NC = 2
_NS = 16
_L = 16
_NW = _NC * _NS

_SHAPE = (2, 8192, 4096)
_N = _SHAPE[0] * _SHAPE[1] * _SHAPE[2]
_PER_W = _N // _NW
_B = 16384
_NBLK = _PER_W // _B

_TBL = 4097
_TBL_PAD = 4112

_MAGIC = np.float32(12582912.0)


def _build_table() -> np.ndarray:
    xs = np.linspace(-8.0, 8.0, num=_TBL).astype(np.float32)
    ys = 1.0 / (1.0 + np.exp(-xs.astype(np.float64)))
    yq = np.rint(ys * 256.0)
    tbl = (yq / 256.0).astype(np.float32)
    out = np.zeros((_TBL_PAD,), dtype=np.float32)
    out[:_TBL] = tbl
    return out


_TABLE = _build_table()


@functools.cache
def _get_lut_kernel():
    mesh = plsc.VectorSubcoreMesh(
        core_axis_name="c", subcore_axis_name="s",
        num_cores=_NC, num_subcores=_NS,
    )

    @functools.partial(
        pl.kernel,
        out_type=jax.ShapeDtypeStruct((_N,), jnp.float32),
        mesh=mesh,
        compiler_params=pltpu.CompilerParams(needs_layout_passes=False),
        scratch_types=[
            pltpu.VMEM((_TBL_PAD,), jnp.float32),
            pltpu.VMEM((_B,), jnp.float32),
            pltpu.VMEM((_B,), jnp.float32),
            pltpu.VMEM((_B,), jnp.float32),
            pltpu.VMEM((_B,), jnp.float32),
            pltpu.SemaphoreType.DMA,
            pltpu.SemaphoreType.DMA,
            pltpu.SemaphoreType.DMA,
            pltpu.SemaphoreType.DMA,
        ],
    )
    def _lut_kernel(x_hbm, tbl_hbm, out_hbm, tbl_v,
                    in0, in1, out0, out1, si0, si1, so0, so1):
        wid = lax.axis_index("s") * _NC + lax.axis_index("c")
        base = wid * _PER_W
        ins, outs = (in0, in1), (out0, out1)
        sis, sos = (si0, si1), (so0, so1)

        def in_copy(g, b):
            off = base + g * _B
            return pltpu.make_async_copy(
                x_hbm.at[pl.ds(off, _B)], ins[b], sis[b])

        def out_copy(g, b):
            off = base + g * _B
            return pltpu.make_async_copy(
                outs[b], out_hbm.at[pl.ds(off, _B)], sos[b])

        def compute(in_v, out_v):
            @plsc.parallel_loop(0, _B // _L, 1, unroll=8)
            def vec(i):
                xv = in_v[pl.ds(i * _L, _L)]
                u = xv * np.float32(256.0) + _MAGIC
                v = u - (_MAGIC - np.float32(2048.0))
                v = jnp.minimum(jnp.maximum(v, np.float32(0.0)),
                                np.float32(4096.0))
                idx = v.astype(jnp.int32)
                out_v[pl.ds(i * _L, _L)] = plsc.load_gather(tbl_v, [idx])

        in_copy(0, 0).start()
        pltpu.sync_copy(tbl_hbm, tbl_v)

        def pair(g2, carry):
            for b in range(2):
                g = g2 * 2 + b

                @pl.when(g + 1 < _NBLK)
                def _():
                    in_copy(g + 1, 1 - b).start()

                in_copy(g, b).wait()

                @pl.when(g >= 2)
                def _():
                    out_copy(g - 2, b).wait()

                compute(ins[b], outs[b])
                out_copy(g, b).start()
            return carry

        lax.fori_loop(0, _NBLK // 2, pair, 0)
        out_copy(_NBLK - 2, 0).wait()
        out_copy(_NBLK - 1, 1).wait()

    return _lut_kernel


def kernel(x):
    y = _get_lut_kernel()(x.reshape(-1), _TABLE)
    return y.reshape(_SHAPE)
